# baseline (speedup 1.0000x reference)
import functools

import jax
import jax.numpy as jnp
from jax import lax
from jax.experimental import pallas as pl
from jax.experimental.pallas import tpu as pltpu
from jax.experimental.pallas import tpu_sc as plsc

N_EXPERTS = 7
K_PROJ = 6
QUANT_TEMP = 0.3
BLK = 512

_NC = 2
_NS = 16
_LANES = 16
_NW = _NC * _NS


def _proj_kernel(x_ref, w_ref, a_ref, rt_ref, logit_ref, q_ref):
    x = x_ref[...]
    w = w_ref[...]
    anchors = a_ref[...]
    inv_2temp = 1.0 / (2.0 * jnp.maximum(rt_ref[0], 0.1))

    zt = jax.lax.dot_general(
        w, x, (((1,), (1,)), ((), ())),
        preferred_element_type=jnp.float32)

    mean = jnp.mean(zt, axis=0, keepdims=True)
    var = jnp.mean((zt - mean) * (zt - mean), axis=0, keepdims=True)
    scale = jnp.sqrt(var) + 1e-6
    qt = jnp.tanh(zt / (QUANT_TEMP * scale))
    q_ref[...] = qt

    dott = jax.lax.dot_general(
        anchors, qt, (((1,), (0,)), ((), ())),
        preferred_element_type=jnp.float32)

    logit_ref[...] = (dott - 6.0) * inv_2temp


def _route_body(chunk, logit_hbm, ew_hbm, loc, out):
    wid = lax.axis_index("s") * _NC + lax.axis_index("c")
    base = wid * chunk
    pltpu.sync_copy(logit_hbm.at[:, pl.ds(base, chunk)], loc)

    for j in range(chunk // _LANES):
        sl = pl.ds(j * _LANES, _LANES)
        l = [loc[e, sl] for e in range(N_EXPERTS)]

        m1 = l[0]
        i1 = jnp.zeros((_LANES,), jnp.int32)
        for e in range(1, N_EXPERTS):
            upd = l[e] > m1
            m1 = jnp.where(upd, l[e], m1)
            i1 = jnp.where(upd, e, i1)
        m2 = jnp.full((_LANES,), -1e30, jnp.float32)
        i2 = jnp.full((_LANES,), N_EXPERTS, jnp.int32)
        for e in range(N_EXPERTS):
            upd = jnp.logical_and(i1 != e, l[e] > m2)
            m2 = jnp.where(upd, l[e], m2)
            i2 = jnp.where(upd, e, i2)

        e2 = jnp.exp(m2 - m1)
        denom = 1.0 + e2
        w1 = 1.0 / denom
        w2 = e2 / denom
        for e in range(N_EXPERTS):
            out[e, sl] = (jnp.where(i1 == e, w1, 0.0)
                          + jnp.where(i2 == e, w2, 0.0))

    pltpu.sync_copy(out, ew_hbm.at[:, pl.ds(base, chunk)])


@jax.jit
def kernel(x, W_proj, routing_temp, expert_anchors):
    B, T, D = x.shape
    n_tok = B * T
    x2 = x.reshape(n_tok, D)
    rt = routing_temp.astype(jnp.float32).reshape(1)

    grid = (n_tok // BLK,)
    logit_t, qt = pl.pallas_call(
        _proj_kernel,
        grid=grid,
        in_specs=[
            pl.BlockSpec((BLK, D), lambda i: (i, 0)),
            pl.BlockSpec((K_PROJ, D), lambda i: (0, 0)),
            pl.BlockSpec((N_EXPERTS, K_PROJ), lambda i: (0, 0)),
            pl.BlockSpec(memory_space=pltpu.SMEM),
        ],
        out_specs=[
            pl.BlockSpec((N_EXPERTS, BLK), lambda i: (0, i)),
            pl.BlockSpec((K_PROJ, BLK), lambda i: (0, i)),
        ],
        out_shape=[
            jax.ShapeDtypeStruct((N_EXPERTS, n_tok), jnp.float32),
            jax.ShapeDtypeStruct((K_PROJ, n_tok), jnp.float32),
        ],
        compiler_params=pltpu.CompilerParams(
            dimension_semantics=("arbitrary",)),
    )(x2, W_proj, expert_anchors, rt)

    chunk = n_tok // _NW
    mesh = plsc.VectorSubcoreMesh(core_axis_name="c", subcore_axis_name="s")
    ewt = pl.kernel(
        functools.partial(_route_body, chunk),
        out_type=jax.ShapeDtypeStruct((N_EXPERTS, n_tok), x.dtype),
        mesh=mesh,
        scratch_types=[
            pltpu.VMEM((N_EXPERTS, chunk), jnp.float32),
            pltpu.VMEM((N_EXPERTS, chunk), jnp.float32),
        ],
    )(logit_t)

    return ewt.T.reshape(B, T, N_EXPERTS), qt.T.reshape(B, T, K_PROJ)


# speedup vs baseline: 1.3426x; 1.3426x over previous
import jax
import jax.numpy as jnp
from jax.experimental import pallas as pl
from jax.experimental.pallas import tpu as pltpu

N_EXPERTS = 7
K_PROJ = 6
QUANT_TEMP = 0.3
BLK = 512


def _router_kernel(x_ref, w_ref, a_ref, rt_ref, ew_ref, q_ref):
    x = x_ref[...]
    w = w_ref[...]
    anchors = a_ref[...]
    inv_2temp = 1.0 / (2.0 * jnp.maximum(rt_ref[0], 0.1))

    zt = jax.lax.dot_general(
        w, x, (((1,), (1,)), ((), ())),
        preferred_element_type=jnp.float32)

    mean = jnp.mean(zt, axis=0, keepdims=True)
    var = jnp.mean((zt - mean) * (zt - mean), axis=0, keepdims=True)
    scale = jnp.sqrt(var) + 1e-6
    qt = jnp.tanh(zt / (QUANT_TEMP * scale))
    q_ref[...] = qt

    dott = jax.lax.dot_general(
        anchors, qt, (((1,), (0,)), ((), ())),
        preferred_element_type=jnp.float32)

    logits = (dott - 6.0) * inv_2temp
    eidx = jax.lax.broadcasted_iota(jnp.int32, logits.shape, 0)

    m1 = jnp.max(logits, axis=0, keepdims=True)
    i1 = jnp.min(jnp.where(logits == m1, eidx, N_EXPERTS),
                 axis=0, keepdims=True)
    is1 = eidx == i1
    rest = jnp.where(is1, -jnp.inf, logits)
    m2 = jnp.max(rest, axis=0, keepdims=True)
    i2 = jnp.min(jnp.where(rest == m2, eidx, N_EXPERTS),
                 axis=0, keepdims=True)

    e2 = jnp.exp(m2 - m1)
    denom = 1.0 + e2
    w1 = 1.0 / denom
    w2 = e2 / denom
    ew_ref[...] = jnp.where(is1, w1, 0.0) + jnp.where(eidx == i2, w2, 0.0)


@jax.jit
def kernel(x, W_proj, routing_temp, expert_anchors):
    B, T, D = x.shape
    n_tok = B * T
    x2 = x.reshape(n_tok, D)
    rt = routing_temp.astype(jnp.float32).reshape(1)

    grid = (n_tok // BLK,)
    ewt, qt = pl.pallas_call(
        _router_kernel,
        grid=grid,
        in_specs=[
            pl.BlockSpec((BLK, D), lambda i: (i, 0)),
            pl.BlockSpec((K_PROJ, D), lambda i: (0, 0)),
            pl.BlockSpec((N_EXPERTS, K_PROJ), lambda i: (0, 0)),
            pl.BlockSpec(memory_space=pltpu.SMEM),
        ],
        out_specs=[
            pl.BlockSpec((N_EXPERTS, BLK), lambda i: (0, i)),
            pl.BlockSpec((K_PROJ, BLK), lambda i: (0, i)),
        ],
        out_shape=[
            jax.ShapeDtypeStruct((N_EXPERTS, n_tok), x.dtype),
            jax.ShapeDtypeStruct((K_PROJ, n_tok), jnp.float32),
        ],
        compiler_params=pltpu.CompilerParams(
            dimension_semantics=("parallel",)),
    )(x2, W_proj, expert_anchors, rt)

    return ewt.T.reshape(B, T, N_EXPERTS), qt.T.reshape(B, T, K_PROJ)
